# baseline (speedup 1.0000x reference)
import functools

import jax
import jax.numpy as jnp
from jax import lax
from jax.experimental import pallas as pl
from jax.experimental.pallas import tpu as pltpu
from jax.experimental.pallas import tpu_sc as plsc

NUM_EMB = 1000000
D = 32
B = 16384
DP = 128

NC = 2
NS = 16
NW = NC * NS

BSPAN = 2048
NBKT = (NUM_EMB + BSPAN - 1) // BSPAN
LASTB = NBKT - 1
LAST_SPAN = 640
CAP = 128
TPW = 16
NCH = B // 16
PAD0 = B
BP = B + TPW * DP

_mesh = plsc.VectorSubcoreMesh(core_axis_name="c", subcore_axis_name="s")


@functools.partial(
    pl.kernel,
    out_type=jax.ShapeDtypeStruct((BP, DP), jnp.float32),
    mesh=_mesh,
    scratch_types=[
        pltpu.VMEM((B,), jnp.int32),
        pltpu.VMEM((16,), jnp.int32),
        pltpu.VMEM((TPW, CAP), jnp.int32),
        pltpu.VMEM((TPW, CAP), jnp.int32),
        pltpu.VMEM((D, BSPAN), jnp.float32),
        pltpu.VMEM((CAP, DP), jnp.float32),
        pltpu.SemaphoreType.DMA,
        pltpu.SemaphoreType.DMA,
        pltpu.SemaphoreType.DMA,
    ],
    compiler_params=pltpu.CompilerParams(
        use_tc_tiling_on_sc=True, needs_layout_passes=False
    ),
)
def _sc_gather(ids_hbm, tabT_hbm, out_hbm, ids_v, fill, idst, post, bbuf, slab,
               semi, semf, semo):
    wid = lax.axis_index("s") * NC + lax.axis_index("c")
    cpi = pltpu.async_copy(ids_hbm, ids_v, semi)

    lanes = lax.iota(jnp.int32, 16)
    fill[...] = jnp.zeros((16,), jnp.int32)
    for t in range(TPW):
        for k in range(CAP // 16):
            plsc.store_scatter(
                post,
                [jnp.full((16,), t, jnp.int32), k * 16 + lanes],
                jnp.full((16,), PAD0 + t * DP, jnp.int32) + k * 16 + lanes,
            )
    cpi.wait()

    def scan_chunk(c, _):
        idv = ids_v[pl.ds(c * 16, 16)]
        bv = lax.shift_right_logical(idv, jnp.int32(11))
        mine = lax.bitwise_and(bv, jnp.int32(31)) == wid
        tv = lax.shift_right_logical(bv, jnp.int32(5))
        cnt, last = plsc.scan_count(bv)
        basev = plsc.load_gather(fill, [tv])
        posv = basev + cnt - 1
        plsc.store_scatter(idst, [tv, posv], idv, mask=mine)
        plsc.store_scatter(post, [tv, posv], c * 16 + lanes, mask=mine)
        plsc.store_scatter(
            fill, [tv], basev + cnt, mask=lax.bitwise_and(mine, last)
        )
        return 0

    lax.fori_loop(0, NCH, scan_chunk, 0, unroll=False)

    fv = fill[...]

    for t in range(TPW):
        b = wid + 32 * t
        n_t = fv[t]

        @pl.when(b <= LASTB)
        def _():
            off = pl.multiple_of(b * BSPAN, 128)
            if t == TPW - 1:
                @pl.when(b == LASTB)
                def _():
                    offl = pl.multiple_of(LASTB * BSPAN, 128)
                    pltpu.async_copy(
                        tabT_hbm.at[:, pl.ds(offl, LAST_SPAN)],
                        bbuf.at[:, pl.ds(0, LAST_SPAN)],
                        semf,
                    ).wait()

                @pl.when(b < LASTB)
                def _():
                    pltpu.async_copy(
                        tabT_hbm.at[:, pl.ds(off, BSPAN)], bbuf, semf
                    ).wait()
            else:
                pltpu.async_copy(
                    tabT_hbm.at[:, pl.ds(off, BSPAN)], bbuf, semf
                ).wait()

            def extract(k, _):
                idb = (idst.at[t])[pl.ds(k * 16, 16)]
                lane_v = lax.bitwise_and(idb, jnp.int32(BSPAN - 1))
                m = (k * 16 + lanes) < n_t
                for j in range(D):
                    j_vec = jnp.full((16,), j, jnp.int32)
                    vals = plsc.load_gather(bbuf, [j_vec, lane_v], mask=m)
                    plsc.store_scatter(
                        slab, [k * 16 + lanes, j_vec], vals, mask=m
                    )
                return 0

            nch = lax.shift_right_logical(n_t + 15, jnp.int32(4))
            lax.fori_loop(0, nch, extract, 0, unroll=False)
            pltpu.async_copy(slab, out_hbm.at[post.at[t]], semo).wait()


def _mm_body(x_ref, w_ref, b_ref, o_ref):
    xs = x_ref[...][:, 0:D]
    o_ref[...] = (
        lax.dot_general(
            w_ref[...], xs, (((1,), (1,)), ((), ())),
            preferred_element_type=jnp.float32,
        )
        + b_ref[...]
    )


_GR = 8
_mm = pl.pallas_call(
    _mm_body,
    out_shape=jax.ShapeDtypeStruct((D, B), jnp.float32),
    grid=(_GR,),
    in_specs=[
        pl.BlockSpec((B // _GR, DP), lambda i: (i, 0)),
        pl.BlockSpec((D, D), lambda i: (0, 0)),
        pl.BlockSpec((D, 1), lambda i: (0, 0)),
    ],
    out_specs=pl.BlockSpec((D, B // _GR), lambda i: (0, i)),
)


def kernel(appearance_ids, table, W, b):
    ids = appearance_ids.astype(jnp.int32)
    emb_p = _sc_gather(ids, table.T)
    outT = _mm(emb_p, W, b.reshape(D, 1))
    return outT.T


# speedup vs baseline: 1.1120x; 1.1120x over previous
import functools

import jax
import jax.numpy as jnp
from jax import lax
from jax.experimental import pallas as pl
from jax.experimental.pallas import tpu as pltpu
from jax.experimental.pallas import tpu_sc as plsc

NUM_EMB = 1000000
D = 32
B = 16384
DP = 128

NC = 2
NS = 16
NW = NC * NS

BSPAN = 2048
HSPAN = 1024
NBKT = (NUM_EMB + BSPAN - 1) // BSPAN
LASTB = NBKT - 1
LAST_SPAN = 640
CAP = 128
TPW = 16
NCH = B // 16
PAD0 = B
BP = B + TPW * DP

_mesh = plsc.VectorSubcoreMesh(core_axis_name="c", subcore_axis_name="s")


@functools.partial(
    pl.kernel,
    out_type=jax.ShapeDtypeStruct((BP, DP), jnp.float32),
    mesh=_mesh,
    scratch_types=[
        pltpu.VMEM((B,), jnp.int32),
        pltpu.VMEM((16,), jnp.int32),
        pltpu.VMEM((TPW, CAP), jnp.int32),
        pltpu.VMEM((TPW, CAP), jnp.int32),
        pltpu.VMEM((2, D, HSPAN), jnp.float32),
        pltpu.VMEM((2, CAP, DP), jnp.float32),
        pltpu.SemaphoreType.DMA,
        pltpu.SemaphoreType.DMA,
        pltpu.SemaphoreType.DMA,
        pltpu.SemaphoreType.DMA,
    ],
    compiler_params=pltpu.CompilerParams(
        use_tc_tiling_on_sc=True, needs_layout_passes=False
    ),
)
def _sc_gather(ids_hbm, tabT_hbm, out_hbm, ids_v, fill, idst, post, bbuf, slab,
               semi, semf0, semf1, semo):
    wid = lax.axis_index("s") * NC + lax.axis_index("c")
    cpi = pltpu.async_copy(ids_hbm, ids_v, semi)

    semf = [semf0, semf1]

    def transfers(s):
        t, h = s >> 1, s & 1
        b = wid + 32 * t
        off = pl.multiple_of(b * BSPAN + h * HSPAN, 128)
        src = tabT_hbm.at[:, pl.ds(off, HSPAN)]
        if t < TPW - 1:
            return [(None, src, bbuf.at[h], semf[h])]
        out = [(b < LASTB, src, bbuf.at[h], semf[h])]
        if h == 0:
            offl = pl.multiple_of(LASTB * BSPAN, 128)
            out.append((
                b == LASTB,
                tabT_hbm.at[:, pl.ds(offl, LAST_SPAN)],
                bbuf.at[0].at[:, pl.ds(0, LAST_SPAN)],
                semf[0],
            ))
        return out

    def fire(s):
        for cond, src, dst, sem in transfers(s):
            if cond is None:
                pltpu.async_copy(src, dst, sem)
            else:
                @pl.when(cond)
                def _(src=src, dst=dst, sem=sem):
                    pltpu.async_copy(src, dst, sem)

    def wait_step(s):
        for cond, src, dst, sem in transfers(s):
            if cond is None:
                pltpu.make_async_copy(src, dst, sem).wait()
            else:
                @pl.when(cond)
                def _(src=src, dst=dst, sem=sem):
                    pltpu.make_async_copy(src, dst, sem).wait()

    lanes = lax.iota(jnp.int32, 16)
    fill[...] = jnp.zeros((16,), jnp.int32)
    def pad_init(c, _):
        flat = c * 16 + lanes
        tv = lax.shift_right_logical(flat, jnp.int32(7))
        cv = lax.bitwise_and(flat, jnp.int32(DP - 1))
        plsc.store_scatter(post, [tv, cv], PAD0 + flat)
        return 0

    lax.fori_loop(0, TPW * CAP // 16, pad_init, 0, unroll=False)
    cpi.wait()
    fire(0)
    fire(1)

    def scan_chunk(c, _):
        idv = ids_v[pl.ds(c * 16, 16)]
        bv = lax.shift_right_logical(idv, jnp.int32(11))
        mine = lax.bitwise_and(bv, jnp.int32(31)) == wid
        tv = lax.shift_right_logical(bv, jnp.int32(5))
        cnt, last = plsc.scan_count(bv)
        basev = plsc.load_gather(fill, [tv])
        posv = basev + cnt - 1
        plsc.store_scatter(idst, [tv, posv], idv, mask=mine)
        plsc.store_scatter(post, [tv, posv], c * 16 + lanes, mask=mine)
        plsc.store_scatter(
            fill, [tv], basev + cnt, mask=lax.bitwise_and(mine, last)
        )
        return 0

    lax.fori_loop(0, NCH, scan_chunk, 0, unroll=False)

    fv = fill[...]
    scd = {}

    def extract_half(t, h):
        n_t = fv[t]
        sp = jnp.full((16,), t & 1, jnp.int32)
        hv = jnp.full((16,), h, jnp.int32)

        def body(k, _):
            idb = (idst.at[t])[pl.ds(k * 16, 16)]
            lane_v = lax.bitwise_and(idb, jnp.int32(BSPAN - 1))
            m = lax.bitwise_and(
                (k * 16 + lanes) < n_t,
                lax.shift_right_logical(lane_v, jnp.int32(10)) == h,
            )
            lane_l = lax.bitwise_and(lane_v, jnp.int32(HSPAN - 1))

            def jbody(j, _):
                j_vec = jnp.full((16,), 0, jnp.int32) + j
                vals = plsc.load_gather(bbuf, [hv, j_vec, lane_l], mask=m)
                plsc.store_scatter(
                    slab, [sp, k * 16 + lanes, j_vec], vals, mask=m
                )
                return 0

            lax.fori_loop(0, D, jbody, 0, unroll=4)
            return 0

        nch = lax.shift_right_logical(n_t + 15, jnp.int32(4))
        lax.fori_loop(0, nch, body, 0, unroll=False)

    for s in range(2 * TPW):
        t, h = s >> 1, s & 1
        wait_step(s)
        if h == 0 and t >= 2:
            scd[t - 2].wait()
        extract_half(t, h)
        if h == 1:
            scd[t] = pltpu.async_copy(
                slab.at[t & 1], out_hbm.at[post.at[t]], semo
            )
        if s + 2 < 2 * TPW:
            fire(s + 2)

    scd[TPW - 2].wait()
    scd[TPW - 1].wait()


def _mm_body(x_ref, w_ref, b_ref, o_ref):
    xs = x_ref[...][:, 0:D]
    o_ref[...] = (
        lax.dot_general(
            w_ref[...], xs, (((1,), (1,)), ((), ())),
            preferred_element_type=jnp.float32,
        )
        + b_ref[...]
    )


_GR = 8
_mm = pl.pallas_call(
    _mm_body,
    out_shape=jax.ShapeDtypeStruct((D, B), jnp.float32),
    grid=(_GR,),
    in_specs=[
        pl.BlockSpec((B // _GR, DP), lambda i: (i, 0)),
        pl.BlockSpec((D, D), lambda i: (0, 0)),
        pl.BlockSpec((D, 1), lambda i: (0, 0)),
    ],
    out_specs=pl.BlockSpec((D, B // _GR), lambda i: (0, i)),
)


def kernel(appearance_ids, table, W, b):
    ids = appearance_ids.astype(jnp.int32)
    emb_p = _sc_gather(ids, table.T)
    outT = _mm(emb_p, W, b.reshape(D, 1))
    return outT.T


# speedup vs baseline: 1.1523x; 1.0363x over previous
import functools

import jax
import jax.numpy as jnp
from jax import lax
from jax.experimental import pallas as pl
from jax.experimental.pallas import tpu as pltpu
from jax.experimental.pallas import tpu_sc as plsc

NUM_EMB = 1000000
D = 32
B = 16384
DP = 128

NC = 2
NS = 16
NW = NC * NS

BSPAN = 2048
HSPAN = 1024
NBKT = (NUM_EMB + BSPAN - 1) // BSPAN
LASTB = NBKT - 1
LAST_SPAN = 640
CAP = 128
TPW = 16
NCH = B // 16
IBLK = 4096
PAD0 = B
BP = B + TPW * DP

_mesh = plsc.VectorSubcoreMesh(core_axis_name="c", subcore_axis_name="s")


@functools.partial(
    pl.kernel,
    out_type=jax.ShapeDtypeStruct((BP, DP), jnp.float32),
    mesh=_mesh,
    scratch_types=[
        pltpu.VMEM((IBLK,), jnp.int32),
        pltpu.VMEM((IBLK,), jnp.int32),
        pltpu.VMEM((16,), jnp.int32),
        pltpu.VMEM((TPW, CAP), jnp.int32),
        pltpu.VMEM((TPW, CAP), jnp.int32),
        pltpu.VMEM((3, D, HSPAN), jnp.float32),
        pltpu.VMEM((CAP, DP), jnp.float32),
        pltpu.SemaphoreType.DMA,
        pltpu.SemaphoreType.DMA,
        pltpu.SemaphoreType.DMA,
        pltpu.SemaphoreType.DMA,
        pltpu.SemaphoreType.DMA,
    ],
    compiler_params=pltpu.CompilerParams(
        use_tc_tiling_on_sc=True, needs_layout_passes=False
    ),
)
def _sc_gather(ids_hbm, tabT_hbm, out_hbm, ids_a, ids_b, fill, idst, post,
               bbuf, slab, semi, semf0, semf1, semf2, semo):
    wid = lax.axis_index("s") * NC + lax.axis_index("c")

    def ids_blk_copy(i):
        return pltpu.make_async_copy(
            ids_hbm.at[pl.ds(i * IBLK, IBLK)], [ids_a, ids_b][i & 1], semi
        )

    ids_blk_copy(0).start()

    semf = [semf0, semf1, semf2]

    def transfers(s):
        t, h = s >> 1, s & 1
        sl = s % 3
        b = wid + 32 * t
        off = pl.multiple_of(b * BSPAN + h * HSPAN, 128)
        src = tabT_hbm.at[:, pl.ds(off, HSPAN)]
        if t < TPW - 1:
            return [(None, src, bbuf.at[sl], semf[sl])]
        out = [(b < LASTB, src, bbuf.at[sl], semf[sl])]
        if h == 0:
            offl = pl.multiple_of(LASTB * BSPAN, 128)
            out.append((
                b == LASTB,
                tabT_hbm.at[:, pl.ds(offl, LAST_SPAN)],
                bbuf.at[sl].at[:, pl.ds(0, LAST_SPAN)],
                semf[sl],
            ))
        return out

    def fire(s):
        for cond, src, dst, sem in transfers(s):
            if cond is None:
                pltpu.async_copy(src, dst, sem)
            else:
                @pl.when(cond)
                def _(src=src, dst=dst, sem=sem):
                    pltpu.async_copy(src, dst, sem)

    def wait_step(s):
        for cond, src, dst, sem in transfers(s):
            if cond is None:
                pltpu.make_async_copy(src, dst, sem).wait()
            else:
                @pl.when(cond)
                def _(src=src, dst=dst, sem=sem):
                    pltpu.make_async_copy(src, dst, sem).wait()

    lanes = lax.iota(jnp.int32, 16)
    fill[...] = jnp.zeros((16,), jnp.int32)
    def pad_init(c, _):
        flat = c * 16 + lanes
        tv = lax.shift_right_logical(flat, jnp.int32(7))
        cv = lax.bitwise_and(flat, jnp.int32(DP - 1))
        plsc.store_scatter(post, [tv, cv], PAD0 + flat)
        return 0

    lax.fori_loop(0, TPW * CAP // 16, pad_init, 0, unroll=False)
    fire(0)
    fire(1)
    fire(2)

    for blk in range(B // IBLK):
        ids_blk_copy(blk).wait()
        if blk + 1 < B // IBLK:
            ids_blk_copy(blk + 1).start()
        blv = [ids_a, ids_b][blk & 1]

        def scan_chunk(k, _, blv=blv, blk=blk):
            c = blk * (IBLK // 16) + k
            idv = blv[pl.ds(k * 16, 16)]
            bv = lax.shift_right_logical(idv, jnp.int32(11))
            mine = lax.bitwise_and(bv, jnp.int32(31)) == wid
            tv = lax.shift_right_logical(bv, jnp.int32(5))
            cnt, last = plsc.scan_count(bv)
            basev = plsc.load_gather(fill, [tv])
            posv = basev + cnt - 1
            plsc.store_scatter(idst, [tv, posv], idv, mask=mine)
            plsc.store_scatter(post, [tv, posv], c * 16 + lanes, mask=mine)
            plsc.store_scatter(
                fill, [tv], basev + cnt, mask=lax.bitwise_and(mine, last)
            )
            return 0

        lax.fori_loop(0, IBLK // 16, scan_chunk, 0, unroll=False)

    fv = fill[...]
    scd = {}

    def extract_half(t, h, sl):
        n_t = fv[t]
        hv = jnp.full((16,), sl, jnp.int32)

        def body(k, _):
            idb = (idst.at[t])[pl.ds(k * 16, 16)]
            lane_v = lax.bitwise_and(idb, jnp.int32(BSPAN - 1))
            m = lax.bitwise_and(
                (k * 16 + lanes) < n_t,
                lax.shift_right_logical(lane_v, jnp.int32(10)) == h,
            )
            lane_l = lax.bitwise_and(lane_v, jnp.int32(HSPAN - 1))

            def jbody(j, _):
                j_vec = jnp.full((16,), 0, jnp.int32) + j
                vals = plsc.load_gather(bbuf, [hv, j_vec, lane_l], mask=m)
                plsc.store_scatter(
                    slab, [k * 16 + lanes, j_vec], vals, mask=m
                )
                return 0

            lax.fori_loop(0, D, jbody, 0, unroll=4)
            return 0

        nch = lax.shift_right_logical(n_t + 15, jnp.int32(4))
        lax.fori_loop(0, nch, body, 0, unroll=False)

    for s in range(2 * TPW):
        t, h = s >> 1, s & 1
        wait_step(s)
        if h == 0 and t >= 1:
            scd[t - 1].wait()
        extract_half(t, h, s % 3)
        if h == 1:
            scd[t] = pltpu.async_copy(
                slab, out_hbm.at[post.at[t]], semo
            )
        if s + 3 < 2 * TPW:
            fire(s + 3)

    scd[TPW - 1].wait()


def _mm_body(x_ref, w_ref, b_ref, o_ref):
    xs = x_ref[...][:, 0:D]
    o_ref[...] = (
        lax.dot_general(
            w_ref[...], xs, (((1,), (1,)), ((), ())),
            preferred_element_type=jnp.float32,
        )
        + b_ref[...]
    )


_GR = 8
_mm = pl.pallas_call(
    _mm_body,
    out_shape=jax.ShapeDtypeStruct((D, B), jnp.float32),
    grid=(_GR,),
    in_specs=[
        pl.BlockSpec((B // _GR, DP), lambda i: (i, 0)),
        pl.BlockSpec((D, D), lambda i: (0, 0)),
        pl.BlockSpec((D, 1), lambda i: (0, 0)),
    ],
    out_specs=pl.BlockSpec((D, B // _GR), lambda i: (0, i)),
)


def kernel(appearance_ids, table, W, b):
    ids = appearance_ids.astype(jnp.int32)
    emb_p = _sc_gather(ids, table.T)
    outT = _mm(emb_p, W, b.reshape(D, 1))
    return outT.T
